# baseline (speedup 1.0000x reference)
import jax
import jax.numpy as jnp
from jax import lax
from jax.experimental import pallas as pl
from jax.experimental.pallas import tpu as pltpu
from jax.experimental.pallas import tpu_sc as plsc

NC = 2
NS = 16
LN = 16

FS = 32


def _mesh():
    return plsc.VectorSubcoreMesh(
        core_axis_name="c", subcore_axis_name="s", num_cores=NC, num_subcores=NS
    )


def _lane_permute(v, idx):
    dn = lax.GatherDimensionNumbers(
        offset_dims=(), collapsed_slice_dims=(0,), start_index_map=(0,)
    )
    return lax.gather(
        v, idx[:, None], dn, (1,), mode=lax.GatherScatterMode.PROMISE_IN_BOUNDS
    )


def _tc_linear(x, wt, b2, block_rows=1000):
    n, d_in = x.shape
    h = wt.shape[1]
    grid = n // block_rows

    def body(x_ref, w_ref, b_ref, o_ref):
        o_ref[...] = (
            jnp.dot(x_ref[...], w_ref[...], preferred_element_type=jnp.float32)
            + b_ref[...]
        )

    return pl.pallas_call(
        body,
        grid=(grid,),
        in_specs=[
            pl.BlockSpec((block_rows, d_in), lambda i: (i, 0)),
            pl.BlockSpec((d_in, h), lambda i: (0, 0)),
            pl.BlockSpec((1, h), lambda i: (0, 0)),
        ],
        out_specs=pl.BlockSpec((block_rows, h), lambda i: (i, 0)),
        out_shape=jax.ShapeDtypeStruct((n, h), jnp.float32),
    )(x, wt, b2)


def _tc_conv(sums, cnt2, h, wlt, wrt, b2, relu, block_rows=1000):
    n, hdim = h.shape
    nsl = sums.shape[0]
    grid = n // block_rows

    def body(s_ref, c_ref, h_ref, wl_ref, wr_ref, b_ref, o_ref):
        cnt = c_ref[:, 0:1] + c_ref[:, 1:2]
        denom = jnp.maximum(cnt, 1.0)
        acc = (
            jnp.dot(h_ref[...], wr_ref[...], preferred_element_type=jnp.float32)
            + b_ref[...]
        )
        for p in range(nsl):
            mean_p = s_ref[p] / denom
            acc += jnp.dot(
                mean_p,
                wl_ref[p * FS : (p + 1) * FS, :],
                preferred_element_type=jnp.float32,
            )
        if relu:
            acc = jnp.maximum(acc, 0.0)
        o_ref[...] = acc

    return pl.pallas_call(
        body,
        grid=(grid,),
        in_specs=[
            pl.BlockSpec((nsl, block_rows, FS), lambda i: (0, i, 0)),
            pl.BlockSpec((block_rows, 2), lambda i: (i, 0)),
            pl.BlockSpec((block_rows, hdim), lambda i: (i, 0)),
            pl.BlockSpec((hdim, hdim), lambda i: (0, 0)),
            pl.BlockSpec((hdim, hdim), lambda i: (0, 0)),
            pl.BlockSpec((1, hdim), lambda i: (0, 0)),
        ],
        out_specs=pl.BlockSpec((block_rows, hdim), lambda i: (i, 0)),
        out_shape=jax.ShapeDtypeStruct((n, hdim), jnp.float32),
    )(sums, cnt2, h, wlt, wrt, b2)


def _sc_cnt(dst2d, acc_n):
    ep = dst2d.shape[0] * 128
    per_w = ep // (NC * NS)
    ch_rows = 8
    n_chunks = per_w // (128 * ch_rows)
    rows_w = per_w // 128
    stripe = acc_n // NS

    def body(dst_ref, out_ref, dstb, ones_v, zbuf, cnt_sh):
        c = lax.axis_index("c")
        s = lax.axis_index("s")
        wid = s * NC + c
        one = jnp.ones((LN,), jnp.float32)
        zk = jnp.zeros((LN,), jnp.float32)

        def ob(i, _):
            ones_v[pl.ds(pl.multiple_of(i * LN, LN), LN)] = one
            return 0

        lax.fori_loop(0, 128 // LN, ob, 0)

        def zb(i, _):
            zbuf[pl.ds(pl.multiple_of(i * LN, LN), LN)] = zk
            return 0

        lax.fori_loop(0, stripe // LN, zb, 0)
        pltpu.sync_copy(zbuf, cnt_sh.at[pl.ds(s * stripe, stripe)])
        plsc.subcore_barrier()

        def chunk(g, _):
            rb = wid * rows_w + g * ch_rows
            pltpu.sync_copy(dst_ref.at[pl.ds(rb, ch_rows)], dstb)
            for j in range(ch_rows):
                pltpu.sync_copy(ones_v, cnt_sh.at[dstb.at[j]], add=True)
            return 0

        lax.fori_loop(0, n_chunks, chunk, 0)
        plsc.subcore_barrier()

        pltpu.sync_copy(
            cnt_sh.at[pl.ds(s * stripe, stripe)],
            out_ref.at[pl.ds(c * acc_n + s * stripe, stripe)],
        )

    fn = pl.kernel(
        body,
        out_type=jax.ShapeDtypeStruct((NC * acc_n,), jnp.float32),
        mesh=_mesh(),
        scratch_types=[
            pltpu.VMEM((ch_rows, 128), jnp.int32),
            pltpu.VMEM((128,), jnp.float32),
            pltpu.VMEM((acc_n // NS,), jnp.float32),
            pltpu.VMEM_SHARED((acc_n,), jnp.float32),
        ],
    )
    return fn(dst2d)


def _sc_segsum(hview, src2d, dst2d, acc_n):
    ep = src2d.shape[0] * 128
    per_s = ep // NS
    sc_rows = 16
    n_super = per_s // (128 * sc_rows)
    n_u = sc_rows // 2
    rows_s = per_s // 128
    stripe = acc_n // NS

    def body(
        hv_ref, src_ref, dst_ref, out_ref,
        srcb, dstb, idxb, rowsA, rowsB,
        acc, gsA, gsB, ssA, ssB,
    ):
        c = lax.axis_index("c")
        s = lax.axis_index("s")
        zk = jnp.zeros((LN,), jnp.float32)
        bufs = ((rowsA, gsA, ssA), (rowsB, gsB, ssB))

        def fire_gather(u, buf):
            rows, gsem, _ = buf
            for j in range(2):
                pltpu.async_copy(
                    hv_ref.at[idxb.at[2 * u + j]],
                    rows.at[pl.ds(j * 128, 128)],
                    gsem,
                )

        def wait_gather(u, buf):
            rows, gsem, _ = buf
            for j in range(2):
                pltpu.make_async_copy(
                    hv_ref.at[idxb.at[2 * u + j]],
                    rows.at[pl.ds(j * 128, 128)],
                    gsem,
                ).wait()

        def fire_scatter(u, buf):
            rows, _, ssem = buf
            for j in range(2):
                pltpu.async_copy(
                    rows.at[pl.ds(j * 128, 128)],
                    acc.at[dstb.at[2 * u + j]],
                    ssem,
                    add=True,
                )

        def wait_scatter(u, buf):
            rows, _, ssem = buf
            for j in range(2):
                pltpu.make_async_copy(
                    rows.at[pl.ds(j * 128, 128)],
                    acc.at[dstb.at[2 * u + j]],
                    ssem,
                ).wait()

        for p_loc in range(2):
            pg = NC * c + p_loc

            def zero_rows(i, _):
                rowsA[i, pl.ds(0, LN)] = zk
                rowsA[i, pl.ds(16, LN)] = zk
                return 0

            lax.fori_loop(0, 256, zero_rows, 0)

            def zstripe(q, _):
                pltpu.sync_copy(
                    rowsA, acc.at[pl.ds(s * stripe + q * 256, 256)]
                )
                return 0

            lax.fori_loop(0, stripe // 256, zstripe, 0)
            if stripe % 256:
                pltpu.sync_copy(
                    rowsA.at[pl.ds(0, stripe % 256)],
                    acc.at[pl.ds(s * stripe + (stripe // 256) * 256, stripe % 256)],
                )
            plsc.subcore_barrier()

            def superchunk(t, _):
                rb = s * rows_s + t * sc_rows
                pltpu.sync_copy(src_ref.at[pl.ds(rb, sc_rows)], srcb)
                pltpu.sync_copy(dst_ref.at[pl.ds(rb, sc_rows)], dstb)

                def idxc(i, _):
                    j = i // 8
                    k = pl.multiple_of((i % 8) * LN, LN)
                    idxb[j, pl.ds(k, LN)] = srcb[j, pl.ds(k, LN)] * 4 + pg
                    return 0

                lax.fori_loop(0, sc_rows * 8, idxc, 0)

                fire_gather(0, bufs[0])
                for u in range(n_u):
                    buf_u = bufs[u % 2]
                    if u + 1 < n_u:
                        buf_n = bufs[(u + 1) % 2]
                        if u >= 1:
                            wait_scatter(u - 1, buf_n)
                        fire_gather(u + 1, buf_n)
                    wait_gather(u, buf_u)
                    fire_scatter(u, buf_u)
                wait_scatter(n_u - 2, bufs[(n_u - 2) % 2])
                wait_scatter(n_u - 1, bufs[(n_u - 1) % 2])
                return 0

            lax.fori_loop(0, n_super, superchunk, 0)
            plsc.subcore_barrier()
            pltpu.sync_copy(
                acc.at[pl.ds(s * stripe, stripe)],
                out_ref.at[pl.ds(pg * acc_n + s * stripe, stripe)],
            )
            plsc.subcore_barrier()

    fn = pl.kernel(
        body,
        out_type=jax.ShapeDtypeStruct((4 * acc_n, FS), jnp.float32),
        mesh=_mesh(),
        scratch_types=[
            pltpu.VMEM((sc_rows, 128), jnp.int32),
            pltpu.VMEM((sc_rows, 128), jnp.int32),
            pltpu.VMEM((sc_rows, 128), jnp.int32),
            pltpu.VMEM((256, FS), jnp.float32),
            pltpu.VMEM((256, FS), jnp.float32),
            pltpu.VMEM_SHARED((acc_n, FS), jnp.float32),
            pltpu.SemaphoreType.DMA,
            pltpu.SemaphoreType.DMA,
            pltpu.SemaphoreType.DMA,
            pltpu.SemaphoreType.DMA,
        ],
        compiler_params=pltpu.CompilerParams(use_tc_tiling_on_sc=False),
    )
    return fn(hview, src2d, dst2d)


def _sc_classifier(h2, src2d, dst2d):
    ep = src2d.shape[0] * 128
    per_w = ep // (NC * NS)
    ch_edges = 128
    n_chunks = per_w // ch_edges
    rows_w = per_w // 128

    def body(
        h2_ref, src_ref, dst_ref, out_ref,
        sball, dball, a0, b0, a1, b1, pb0, pb1,
        gs0, gs1, ps0, ps1,
    ):
        c = lax.axis_index("c")
        s = lax.axis_index("s")
        wid = s * NC + c
        ramp = lax.iota(jnp.int32, LN)
        zk = jnp.zeros((LN,), jnp.float32)
        bufs = ((a0, b0, pb0, gs0, ps0), (a1, b1, pb1, gs1, ps1))

        pltpu.sync_copy(src_ref.at[pl.ds(wid * rows_w, rows_w)], sball)
        pltpu.sync_copy(dst_ref.at[pl.ds(wid * rows_w, rows_w)], dball)

        def fire(g, buf):
            abuf, bbuf, _, gsem, _ = buf
            pltpu.async_copy(h2_ref.at[sball.at[g]], abuf, gsem)
            pltpu.async_copy(h2_ref.at[dball.at[g]], bbuf, gsem)

        def wait_compute(g, buf, first):
            abuf, bbuf, pb, gsem, psem = buf
            pltpu.make_async_copy(h2_ref.at[sball.at[g]], abuf, gsem).wait()
            pltpu.make_async_copy(h2_ref.at[dball.at[g]], bbuf, gsem).wait()

            @pl.when(jnp.logical_not(first))
            def _():
                pltpu.make_async_copy(
                    pb, out_ref.at[pl.ds(wid * per_w + (g - 2) * ch_edges, ch_edges)],
                    psem,
                ).wait()

            final_idx = ((ramp & 1) << 3) | ((ramp & 2) << 1)
            for jj in range(ch_edges // 16):

                def quad(q, res):
                    row = jj * 16 + q * 4
                    accs = []
                    for de in range(4):
                        acc = zk
                        for k in range(8):
                            ko = pl.multiple_of(k * LN, LN)
                            acc = acc + (
                                abuf[row + de, pl.ds(ko, LN)]
                                * bbuf[row + de, pl.ds(ko, LN)]
                            )
                        accs.append(acc)
                    a0, a1, a2, a3 = accs
                    t01 = jnp.where(
                        ramp < 8,
                        a0 + _lane_permute(a0, ramp ^ 8),
                        a1 + _lane_permute(a1, ramp ^ 8),
                    )
                    t23 = jnp.where(
                        ramp < 8,
                        a2 + _lane_permute(a2, ramp ^ 8),
                        a3 + _lane_permute(a3, ramp ^ 8),
                    )
                    u = t01 + _lane_permute(t01, ramp ^ 4)
                    v = t23 + _lane_permute(t23, ramp ^ 4)
                    comb = jnp.where((ramp & 4) == 0, u, v)
                    w = comb + _lane_permute(comb, ramp ^ 2)
                    w = w + _lane_permute(w, ramp ^ 1)
                    p = _lane_permute(w, final_idx)
                    return jnp.where((ramp >> 2) == q, p, res)

                res = lax.fori_loop(0, 4, quad, zk)
                pb[pl.ds(jj * 16, LN)] = res

            pltpu.async_copy(
                pb, out_ref.at[pl.ds(wid * per_w + g * ch_edges, ch_edges)], psem
            )

        fire(0, bufs[0])

        def pair(gp, _):
            g0 = 2 * gp
            fire(g0 + 1, bufs[1])
            wait_compute(g0, bufs[0], gp == 0)

            @pl.when(gp < n_chunks // 2 - 1)
            def _():
                fire(g0 + 2, bufs[0])

            wait_compute(g0 + 1, bufs[1], gp == 0)
            return 0

        lax.fori_loop(0, n_chunks // 2, pair, 0)
        for last, buf in ((n_chunks - 2, bufs[0]), (n_chunks - 1, bufs[1])):
            pltpu.make_async_copy(
                buf[2],
                out_ref.at[pl.ds(wid * per_w + last * ch_edges, ch_edges)],
                buf[4],
            ).wait()

    fn = pl.kernel(
        body,
        out_type=jax.ShapeDtypeStruct((ep,), jnp.float32),
        mesh=_mesh(),
        scratch_types=[
            pltpu.VMEM((rows_w, 128), jnp.int32),
            pltpu.VMEM((rows_w, 128), jnp.int32),
            pltpu.VMEM((128, 128), jnp.float32),
            pltpu.VMEM((128, 128), jnp.float32),
            pltpu.VMEM((128, 128), jnp.float32),
            pltpu.VMEM((128, 128), jnp.float32),
            pltpu.VMEM((128,), jnp.float32),
            pltpu.VMEM((128,), jnp.float32),
            pltpu.SemaphoreType.DMA,
            pltpu.SemaphoreType.DMA,
            pltpu.SemaphoreType.DMA,
            pltpu.SemaphoreType.DMA,
        ],
    )
    return fn(h2, src2d, dst2d)


def kernel(x, edge_index, W_lin, b_lin, W1l, b1l, W1r, W2l, b2l, W2r):
    n, _ = x.shape
    e = edge_index.shape[1]
    h_dim = W_lin.shape[0]

    ep = -(-e // 32768) * 32768
    acc_n = -(-(n + 1) // 2048) * 2048
    pad = ep - e

    src = edge_index[0]
    dst = edge_index[1]
    srcp = jnp.concatenate([src, jnp.zeros((pad,), src.dtype)]).reshape(-1, 128)
    dstp_sc = jnp.concatenate(
        [dst, jnp.full((pad,), n, dst.dtype)]
    ).reshape(-1, 128)
    dstp_cl = jnp.concatenate([dst, jnp.zeros((pad,), dst.dtype)]).reshape(-1, 128)

    cnt_parts = _sc_cnt(dstp_sc, acc_n)
    cnt2 = cnt_parts.reshape(2, acc_n).T

    h = _tc_linear(x, W_lin.T, b_lin.reshape(1, h_dim))
    s1 = _sc_segsum(h.reshape(n * 4, FS), srcp, dstp_sc, acc_n)
    h1 = _tc_conv(
        s1.reshape(4, acc_n, FS), cnt2, h, W1l.T, W1r.T, b1l.reshape(1, h_dim), True
    )
    s2 = _sc_segsum(h1.reshape(n * 4, FS), srcp, dstp_sc, acc_n)
    h2 = _tc_conv(
        s2.reshape(4, acc_n, FS), cnt2, h1, W2l.T, W2r.T, b2l.reshape(1, h_dim), False
    )
    pred = _sc_classifier(h2, srcp, dstp_cl)[:e]
    return (pred, h2)


# speedup vs baseline: 1.0217x; 1.0217x over previous
import jax
import jax.numpy as jnp
from jax import lax
from jax.experimental import pallas as pl
from jax.experimental.pallas import tpu as pltpu
from jax.experimental.pallas import tpu_sc as plsc

NC = 2
NS = 16
LN = 16

FS = 32


def _mesh():
    return plsc.VectorSubcoreMesh(
        core_axis_name="c", subcore_axis_name="s", num_cores=NC, num_subcores=NS
    )


def _lane_permute(v, idx):
    dn = lax.GatherDimensionNumbers(
        offset_dims=(), collapsed_slice_dims=(0,), start_index_map=(0,)
    )
    return lax.gather(
        v, idx[:, None], dn, (1,), mode=lax.GatherScatterMode.PROMISE_IN_BOUNDS
    )


def _tc_linear(x, wt, b2, block_rows=1000):
    n, d_in = x.shape
    h = wt.shape[1]
    grid = n // block_rows

    def body(x_ref, w_ref, b_ref, o_ref):
        o_ref[...] = (
            jnp.dot(x_ref[...], w_ref[...], preferred_element_type=jnp.float32)
            + b_ref[...]
        )

    return pl.pallas_call(
        body,
        grid=(grid,),
        in_specs=[
            pl.BlockSpec((block_rows, d_in), lambda i: (i, 0)),
            pl.BlockSpec((d_in, h), lambda i: (0, 0)),
            pl.BlockSpec((1, h), lambda i: (0, 0)),
        ],
        out_specs=pl.BlockSpec((block_rows, h), lambda i: (i, 0)),
        out_shape=jax.ShapeDtypeStruct((n, h), jnp.float32),
    )(x, wt, b2)


def _tc_conv(sums, cnt2, h, wlt, wrt, b2, relu, block_rows=1000):
    n, hdim = h.shape
    nsl = sums.shape[0]
    grid = n // block_rows

    def body(s_ref, c_ref, h_ref, wl_ref, wr_ref, b_ref, o_ref):
        cnt = c_ref[:, 0:1] + c_ref[:, 1:2]
        denom = jnp.maximum(cnt, 1.0)
        acc = (
            jnp.dot(h_ref[...], wr_ref[...], preferred_element_type=jnp.float32)
            + b_ref[...]
        )
        for p in range(nsl):
            mean_p = s_ref[p] / denom
            acc += jnp.dot(
                mean_p,
                wl_ref[p * FS : (p + 1) * FS, :],
                preferred_element_type=jnp.float32,
            )
        if relu:
            acc = jnp.maximum(acc, 0.0)
        o_ref[...] = acc

    return pl.pallas_call(
        body,
        grid=(grid,),
        in_specs=[
            pl.BlockSpec((nsl, block_rows, FS), lambda i: (0, i, 0)),
            pl.BlockSpec((block_rows, 2), lambda i: (i, 0)),
            pl.BlockSpec((block_rows, hdim), lambda i: (i, 0)),
            pl.BlockSpec((hdim, hdim), lambda i: (0, 0)),
            pl.BlockSpec((hdim, hdim), lambda i: (0, 0)),
            pl.BlockSpec((1, hdim), lambda i: (0, 0)),
        ],
        out_specs=pl.BlockSpec((block_rows, hdim), lambda i: (i, 0)),
        out_shape=jax.ShapeDtypeStruct((n, hdim), jnp.float32),
    )(sums, cnt2, h, wlt, wrt, b2)


def _sc_cnt(dst2d, acc_n):
    ep = dst2d.shape[0] * 128
    per_w = ep // (NC * NS)
    ch_rows = 8
    n_chunks = per_w // (128 * ch_rows)
    rows_w = per_w // 128
    stripe = acc_n // NS

    def body(dst_ref, out_ref, dstb, ones_v, zbuf, cnt_sh):
        c = lax.axis_index("c")
        s = lax.axis_index("s")
        wid = s * NC + c
        one = jnp.ones((LN,), jnp.float32)
        zk = jnp.zeros((LN,), jnp.float32)

        def ob(i, _):
            ones_v[pl.ds(pl.multiple_of(i * LN, LN), LN)] = one
            return 0

        lax.fori_loop(0, 128 // LN, ob, 0)

        def zb(i, _):
            zbuf[pl.ds(pl.multiple_of(i * LN, LN), LN)] = zk
            return 0

        lax.fori_loop(0, stripe // LN, zb, 0)
        pltpu.sync_copy(zbuf, cnt_sh.at[pl.ds(s * stripe, stripe)])
        plsc.subcore_barrier()

        def chunk(g, _):
            rb = wid * rows_w + g * ch_rows
            pltpu.sync_copy(dst_ref.at[pl.ds(rb, ch_rows)], dstb)
            for j in range(ch_rows):
                pltpu.sync_copy(ones_v, cnt_sh.at[dstb.at[j]], add=True)
            return 0

        lax.fori_loop(0, n_chunks, chunk, 0)
        plsc.subcore_barrier()

        pltpu.sync_copy(
            cnt_sh.at[pl.ds(s * stripe, stripe)],
            out_ref.at[pl.ds(c * acc_n + s * stripe, stripe)],
        )

    fn = pl.kernel(
        body,
        out_type=jax.ShapeDtypeStruct((NC * acc_n,), jnp.float32),
        mesh=_mesh(),
        scratch_types=[
            pltpu.VMEM((ch_rows, 128), jnp.int32),
            pltpu.VMEM((128,), jnp.float32),
            pltpu.VMEM((acc_n // NS,), jnp.float32),
            pltpu.VMEM_SHARED((acc_n,), jnp.float32),
        ],
    )
    return fn(dst2d)


def _sc_segsum(hview, src2d, dst2d, acc_n):
    ep = src2d.shape[0] * 128
    per_s = ep // NS
    sc_rows = 16
    n_super = per_s // (128 * sc_rows)
    n_u = sc_rows // 2
    rows_s = per_s // 128
    stripe = acc_n // NS

    def body(
        hv_ref, src_ref, dst_ref, out_ref,
        srcb, dstb, idxb, rowsA, rowsB,
        acc, gsA, gsB, ssA, ssB,
    ):
        c = lax.axis_index("c")
        s = lax.axis_index("s")
        zk = jnp.zeros((LN,), jnp.float32)
        bufs = ((rowsA, gsA, ssA), (rowsB, gsB, ssB))

        def fire_gather(u, buf):
            rows, gsem, _ = buf
            for j in range(2):
                pltpu.async_copy(
                    hv_ref.at[idxb.at[2 * u + j]],
                    rows.at[pl.ds(j * 128, 128)],
                    gsem,
                )

        def wait_gather(u, buf):
            rows, gsem, _ = buf
            for j in range(2):
                pltpu.make_async_copy(
                    hv_ref.at[idxb.at[2 * u + j]],
                    rows.at[pl.ds(j * 128, 128)],
                    gsem,
                ).wait()

        def fire_scatter(u, buf):
            rows, _, ssem = buf
            for j in range(2):
                pltpu.async_copy(
                    rows.at[pl.ds(j * 128, 128)],
                    acc.at[dstb.at[2 * u + j]],
                    ssem,
                    add=True,
                )

        def wait_scatter(u, buf):
            rows, _, ssem = buf
            for j in range(2):
                pltpu.make_async_copy(
                    rows.at[pl.ds(j * 128, 128)],
                    acc.at[dstb.at[2 * u + j]],
                    ssem,
                ).wait()

        for p_loc in range(2):
            pg = NC * c + p_loc

            def zero_rows(i, _):
                rowsA[i, pl.ds(0, LN)] = zk
                rowsA[i, pl.ds(16, LN)] = zk
                return 0

            lax.fori_loop(0, 256, zero_rows, 0)

            def zstripe(q, _):
                pltpu.sync_copy(
                    rowsA, acc.at[pl.ds(s * stripe + q * 256, 256)]
                )
                return 0

            lax.fori_loop(0, stripe // 256, zstripe, 0)
            if stripe % 256:
                pltpu.sync_copy(
                    rowsA.at[pl.ds(0, stripe % 256)],
                    acc.at[pl.ds(s * stripe + (stripe // 256) * 256, stripe % 256)],
                )
            plsc.subcore_barrier()

            def superchunk(t, _):
                rb = s * rows_s + t * sc_rows
                pltpu.sync_copy(src_ref.at[pl.ds(rb, sc_rows)], srcb)
                pltpu.sync_copy(dst_ref.at[pl.ds(rb, sc_rows)], dstb)

                def idxc(i, _):
                    j = i // 8
                    k = pl.multiple_of((i % 8) * LN, LN)
                    idxb[j, pl.ds(k, LN)] = srcb[j, pl.ds(k, LN)] * 4 + pg
                    return 0

                lax.fori_loop(0, sc_rows * 8, idxc, 0)

                fire_gather(0, bufs[0])
                for u in range(n_u):
                    buf_u = bufs[u % 2]
                    if u + 1 < n_u:
                        buf_n = bufs[(u + 1) % 2]
                        if u >= 1:
                            wait_scatter(u - 1, buf_n)
                        fire_gather(u + 1, buf_n)
                    wait_gather(u, buf_u)
                    fire_scatter(u, buf_u)
                wait_scatter(n_u - 2, bufs[(n_u - 2) % 2])
                wait_scatter(n_u - 1, bufs[(n_u - 1) % 2])
                return 0

            lax.fori_loop(0, n_super, superchunk, 0)
            plsc.subcore_barrier()
            pltpu.sync_copy(
                acc.at[pl.ds(s * stripe, stripe)],
                out_ref.at[pl.ds(pg * acc_n + s * stripe, stripe)],
            )
            plsc.subcore_barrier()

    fn = pl.kernel(
        body,
        out_type=jax.ShapeDtypeStruct((4 * acc_n, FS), jnp.float32),
        mesh=_mesh(),
        scratch_types=[
            pltpu.VMEM((sc_rows, 128), jnp.int32),
            pltpu.VMEM((sc_rows, 128), jnp.int32),
            pltpu.VMEM((sc_rows, 128), jnp.int32),
            pltpu.VMEM((256, FS), jnp.float32),
            pltpu.VMEM((256, FS), jnp.float32),
            pltpu.VMEM_SHARED((acc_n, FS), jnp.float32),
            pltpu.SemaphoreType.DMA,
            pltpu.SemaphoreType.DMA,
            pltpu.SemaphoreType.DMA,
            pltpu.SemaphoreType.DMA,
        ],
        compiler_params=pltpu.CompilerParams(use_tc_tiling_on_sc=False),
    )
    return fn(hview, src2d, dst2d)


def _sc_classifier(h2, src2d, dst2d):
    ep = src2d.shape[0] * 128
    per_w = ep // (NC * NS)
    ch_edges = 128
    n_chunks = per_w // ch_edges
    rows_w = per_w // 128

    def body(
        h2_ref, src_ref, dst_ref, out_ref,
        sball, dball, a0, b0, a1, b1, pb0, pb1,
        gs0, gs1, ps0, ps1,
    ):
        c = lax.axis_index("c")
        s = lax.axis_index("s")
        wid = s * NC + c
        ramp = lax.iota(jnp.int32, LN)
        zk = jnp.zeros((LN,), jnp.float32)
        bufs = ((a0, b0, pb0, gs0, ps0), (a1, b1, pb1, gs1, ps1))

        pltpu.sync_copy(src_ref.at[pl.ds(wid * rows_w, rows_w)], sball)
        pltpu.sync_copy(dst_ref.at[pl.ds(wid * rows_w, rows_w)], dball)

        def fire(g, buf):
            abuf, bbuf, _, gsem, _ = buf
            pltpu.async_copy(h2_ref.at[sball.at[g]], abuf, gsem)
            pltpu.async_copy(h2_ref.at[dball.at[g]], bbuf, gsem)

        def wait_compute(g, buf, first):
            abuf, bbuf, pb, gsem, psem = buf
            pltpu.make_async_copy(h2_ref.at[sball.at[g]], abuf, gsem).wait()
            pltpu.make_async_copy(h2_ref.at[dball.at[g]], bbuf, gsem).wait()

            @pl.when(jnp.logical_not(first))
            def _():
                pltpu.make_async_copy(
                    pb, out_ref.at[pl.ds(wid * per_w + (g - 2) * ch_edges, ch_edges)],
                    psem,
                ).wait()

            final_idx = ((ramp & 1) << 3) | ((ramp & 2) << 1)
            for jj in range(ch_edges // 16):

                def quad(q, res):
                    row = jj * 16 + q * 4
                    accs = []
                    for de in range(4):
                        acc = zk
                        for k in range(8):
                            ko = pl.multiple_of(k * LN, LN)
                            acc = acc + (
                                abuf[row + de, pl.ds(ko, LN)]
                                * bbuf[row + de, pl.ds(ko, LN)]
                            )
                        accs.append(acc)
                    a0, a1, a2, a3 = accs
                    t01 = jnp.where(
                        ramp < 8,
                        a0 + _lane_permute(a0, ramp ^ 8),
                        a1 + _lane_permute(a1, ramp ^ 8),
                    )
                    t23 = jnp.where(
                        ramp < 8,
                        a2 + _lane_permute(a2, ramp ^ 8),
                        a3 + _lane_permute(a3, ramp ^ 8),
                    )
                    u = t01 + _lane_permute(t01, ramp ^ 4)
                    v = t23 + _lane_permute(t23, ramp ^ 4)
                    comb = jnp.where((ramp & 4) == 0, u, v)
                    w = comb + _lane_permute(comb, ramp ^ 2)
                    w = w + _lane_permute(w, ramp ^ 1)
                    p = _lane_permute(w, final_idx)
                    return jnp.where((ramp >> 2) == q, p, res)

                res = lax.fori_loop(0, 4, quad, zk)
                pb[pl.ds(jj * 16, LN)] = res

            pltpu.async_copy(
                pb, out_ref.at[pl.ds(wid * per_w + g * ch_edges, ch_edges)], psem
            )

        fire(0, bufs[0])

        def pair(gp, _):
            g0 = 2 * gp
            fire(g0 + 1, bufs[1])
            wait_compute(g0, bufs[0], gp == 0)

            @pl.when(gp < n_chunks // 2 - 1)
            def _():
                fire(g0 + 2, bufs[0])

            wait_compute(g0 + 1, bufs[1], gp == 0)
            return 0

        lax.fori_loop(0, n_chunks // 2, pair, 0)
        for last, buf in ((n_chunks - 2, bufs[0]), (n_chunks - 1, bufs[1])):
            pltpu.make_async_copy(
                buf[2],
                out_ref.at[pl.ds(wid * per_w + last * ch_edges, ch_edges)],
                buf[4],
            ).wait()

    fn = pl.kernel(
        body,
        out_type=jax.ShapeDtypeStruct((ep,), jnp.float32),
        mesh=_mesh(),
        scratch_types=[
            pltpu.VMEM((rows_w, 128), jnp.int32),
            pltpu.VMEM((rows_w, 128), jnp.int32),
            pltpu.VMEM((128, 128), jnp.float32),
            pltpu.VMEM((128, 128), jnp.float32),
            pltpu.VMEM((128, 128), jnp.float32),
            pltpu.VMEM((128, 128), jnp.float32),
            pltpu.VMEM((128,), jnp.float32),
            pltpu.VMEM((128,), jnp.float32),
            pltpu.SemaphoreType.DMA,
            pltpu.SemaphoreType.DMA,
            pltpu.SemaphoreType.DMA,
            pltpu.SemaphoreType.DMA,
        ],
        compiler_params=pltpu.CompilerParams(use_tc_tiling_on_sc=False),
    )
    return fn(h2, src2d, dst2d)


def kernel(x, edge_index, W_lin, b_lin, W1l, b1l, W1r, W2l, b2l, W2r):
    n, _ = x.shape
    e = edge_index.shape[1]
    h_dim = W_lin.shape[0]

    ep = -(-e // 32768) * 32768
    acc_n = -(-(n + 1) // 2048) * 2048
    pad = ep - e

    src = edge_index[0]
    dst = edge_index[1]
    srcp = jnp.concatenate([src, jnp.zeros((pad,), src.dtype)]).reshape(-1, 128)
    dstp_sc = jnp.concatenate(
        [dst, jnp.full((pad,), n, dst.dtype)]
    ).reshape(-1, 128)
    dstp_cl = jnp.concatenate([dst, jnp.zeros((pad,), dst.dtype)]).reshape(-1, 128)

    cnt_parts = _sc_cnt(dstp_sc, acc_n)
    cnt2 = cnt_parts.reshape(2, acc_n).T

    h = _tc_linear(x, W_lin.T, b_lin.reshape(1, h_dim))
    s1 = _sc_segsum(h.reshape(n * 4, FS), srcp, dstp_sc, acc_n)
    h1 = _tc_conv(
        s1.reshape(4, acc_n, FS), cnt2, h, W1l.T, W1r.T, b1l.reshape(1, h_dim), True
    )
    s2 = _sc_segsum(h1.reshape(n * 4, FS), srcp, dstp_sc, acc_n)
    h2 = _tc_conv(
        s2.reshape(4, acc_n, FS), cnt2, h1, W2l.T, W2r.T, b2l.reshape(1, h_dim), False
    )
    pred = _sc_classifier(h2, srcp, dstp_cl)[:e]
    return (pred, h2)
